# Initial kernel scaffold; baseline (speedup 1.0000x reference)
import jax
import jax.numpy as jnp
from jax.experimental import pallas as pl


def kernel(batch_size, img_feature_map, word_features, W1, W2, W3, b3, Wa, ba):
    raise NotImplementedError("write your pallas kernel here")


# speedup vs baseline: 3.3729x; 3.3729x over previous
import jax
import jax.numpy as jnp
from jax.experimental import pallas as pl
from jax.experimental.pallas import tpu as pltpu

_CC = 16


def _sem_kernel(fmap_ref, word_ref, W1_ref, W2_ref, W3_ref, Wa_ref, out_ref):
    fmap = fmap_ref[0]
    f_wh = jax.lax.dot_general(
        fmap, W1_ref[...], (((1,), (1,)), ((), ())),
        preferred_element_type=jnp.float32)
    f_wd = jax.lax.dot_general(
        word_ref[...], W2_ref[...], (((1,), (1,)), ((), ())),
        preferred_element_type=jnp.float32)
    v = jax.lax.dot_general(
        Wa_ref[...], W3_ref[...], (((1,), (0,)), ((), ())),
        preferred_element_type=jnp.float32)

    C = f_wd.shape[0]
    cols = []
    for c0 in range(0, C, _CC):
        fd = f_wd[c0:c0 + _CC]
        t = jnp.tanh(f_wh[:, None, :] * fd[None, :, :])
        cols.append(jnp.sum(t * v[0][None, None, :], axis=2))
    coef = jnp.concatenate(cols, axis=1)

    coef = coef - jnp.max(coef, axis=0, keepdims=True)
    e = jnp.exp(coef)
    coef = e / jnp.sum(e, axis=0, keepdims=True)

    out_ref[0] = jax.lax.dot_general(
        coef, fmap, (((0,), (0,)), ((), ())),
        preferred_element_type=jnp.float32)


def kernel(batch_size, img_feature_map, word_features, W1, W2, W3, b3, Wa, ba):
    Bn, D, H, W = img_feature_map.shape
    S = H * W
    fmap = jnp.transpose(img_feature_map, (0, 2, 3, 1)).reshape(Bn, S, D)
    C, DW = word_features.shape
    M = W1.shape[0]
    return pl.pallas_call(
        _sem_kernel,
        grid=(Bn,),
        in_specs=[
            pl.BlockSpec((1, S, D), lambda b: (b, 0, 0)),
            pl.BlockSpec((C, DW), lambda b: (0, 0)),
            pl.BlockSpec((M, D), lambda b: (0, 0)),
            pl.BlockSpec((M, DW), lambda b: (0, 0)),
            pl.BlockSpec((M, M), lambda b: (0, 0)),
            pl.BlockSpec((1, M), lambda b: (0, 0)),
        ],
        out_specs=pl.BlockSpec((1, C, D), lambda b: (b, 0, 0)),
        out_shape=jax.ShapeDtypeStruct((Bn, C, D), jnp.float32),
        compiler_params=pltpu.CompilerParams(
            dimension_semantics=("parallel",),
            vmem_limit_bytes=56 * 1024 * 1024,
        ),
    )(fmap, word_features, W1, W2, W3, Wa)


# speedup vs baseline: 3.5023x; 1.0384x over previous
import jax
import jax.numpy as jnp
from jax.experimental import pallas as pl
from jax.experimental.pallas import tpu as pltpu

_CC = 16


def _prep_kernel(word_ref, W2_ref, W3_ref, Wa_ref, fwd_ref, v_ref):
    fwd_ref[...] = jax.lax.dot_general(
        word_ref[...], W2_ref[...], (((1,), (1,)), ((), ())),
        preferred_element_type=jnp.float32)
    v_ref[...] = jax.lax.dot_general(
        Wa_ref[...], W3_ref[...], (((1,), (0,)), ((), ())),
        preferred_element_type=jnp.float32)


def _main_kernel(fmapd_ref, W1_ref, fdflat_ref, v_ref, out_ref):
    fmap_d = fmapd_ref[0]
    f_wh = jax.lax.dot_general(
        fmap_d, W1_ref[...], (((0,), (1,)), ((), ())),
        preferred_element_type=jnp.float32)

    M = v_ref.shape[1]
    CT = fdflat_ref.shape[1] // M
    fwh_rep = pltpu.repeat(f_wh, _CC, axis=1)
    vrow = v_ref[...]

    cols = []
    for j in range(0, CT, _CC):
        fd_j = fdflat_ref[...][:, j * M:(j + _CC) * M]
        t = jnp.tanh(fwh_rep * fd_j)
        for k in range(_CC):
            w = t[:, k * M:(k + 1) * M] * vrow
            cols.append(jnp.sum(w, axis=1, keepdims=True))
    coef = jnp.concatenate(cols, axis=1)

    coef = coef - jnp.max(coef, axis=0, keepdims=True)
    e = jnp.exp(coef)
    coef = e / jnp.sum(e, axis=0, keepdims=True)

    out_ref[0] = jax.lax.dot_general(
        coef, fmap_d, (((0,), (1,)), ((), ())),
        preferred_element_type=jnp.float32)


def kernel(batch_size, img_feature_map, word_features, W1, W2, W3, b3, Wa, ba):
    Bn, D, H, W = img_feature_map.shape
    S = H * W
    fmap_d = img_feature_map.reshape(Bn, D, S)
    C, DW = word_features.shape
    M = W1.shape[0]

    f_wd, v = pl.pallas_call(
        _prep_kernel,
        out_shape=(jax.ShapeDtypeStruct((C, M), jnp.float32),
                   jax.ShapeDtypeStruct((1, M), jnp.float32)),
    )(word_features, W2, W3, Wa)
    fd_flat = f_wd.reshape(1, C * M)

    return pl.pallas_call(
        _main_kernel,
        grid=(Bn,),
        in_specs=[
            pl.BlockSpec((1, D, S), lambda b: (b, 0, 0)),
            pl.BlockSpec((M, D), lambda b: (0, 0)),
            pl.BlockSpec((1, C * M), lambda b: (0, 0)),
            pl.BlockSpec((1, M), lambda b: (0, 0)),
        ],
        out_specs=pl.BlockSpec((1, C, D), lambda b: (b, 0, 0)),
        out_shape=jax.ShapeDtypeStruct((Bn, C, D), jnp.float32),
        compiler_params=pltpu.CompilerParams(
            dimension_semantics=("arbitrary",),
            vmem_limit_bytes=56 * 1024 * 1024,
        ),
    )(fmap_d, W1, fd_flat, v)
